# baseline (speedup 1.0000x reference)
import functools

import jax
import jax.numpy as jnp
from jax import lax
from jax.experimental import pallas as pl
from jax.experimental.pallas import tpu as pltpu
from jax.experimental.pallas import tpu_sc as plsc

_B = 4096
_NF = 26
_VOCAB = 100000
_D = 64
_CONT = 13
_OUT_ROWS = _CONT + _NF * _D
_NROWS = _NF * _D

_NC = 2
_NS = 16
_NW = _NC * _NS
_NPAIR = _NROWS // (2 * _NW)

_CHUNK = 16768
_NCHUNK = 6
_LAST = 16128
_ALIGNED = 5 * _CHUNK + _LAST
_TAIL = _VOCAB - _ALIGNED
_GROUPS = _B // 16
_LENS = (_CHUNK,) * 5 + (_LAST,)


@functools.partial(
    pl.kernel,
    out_type=jax.ShapeDtypeStruct((_OUT_ROWS, _B), jnp.float32),
    mesh=plsc.VectorSubcoreMesh(core_axis_name="c", subcore_axis_name="s"),
    scratch_types=[
        pltpu.VMEM((2, _CHUNK), jnp.float32),
        pltpu.VMEM((2, _CHUNK), jnp.float32),
        pltpu.VMEM((2, _CHUNK), jnp.float32),
        pltpu.VMEM((_B,), jnp.int32),
        pltpu.VMEM((_B,), jnp.int32),
        pltpu.VMEM((_TAIL,), jnp.float32),
        pltpu.VMEM((_TAIL,), jnp.float32),
        pltpu.VMEM((_TAIL,), jnp.float32),
        pltpu.VMEM((_TAIL,), jnp.float32),
        pltpu.VMEM((_B,), jnp.float32),
        pltpu.VMEM((_B,), jnp.float32),
        pltpu.SemaphoreType.DMA,
        pltpu.SemaphoreType.DMA,
        pltpu.SemaphoreType.DMA,
        pltpu.SemaphoreType.DMA,
        pltpu.SemaphoreType.DMA,
    ],
    compiler_params=pltpu.CompilerParams(
        needs_layout_passes=False, disable_bounds_checks=True,
        disable_semaphore_checks=True),
)
def _embed(tbl_hbm, idx_hbm, cont_hbm, out_hbm, bufa, bufb, bufc,
           idx0_v, idx1_v, t00, t01, t10, t11, row0_v, row1_v,
           sema, semb, semc, semi, semo):
    wid = lax.axis_index("s") * _NC + lax.axis_index("c")
    a = wid >> 2
    j = wid & 3
    rows = (row0_v, row1_v)
    idxs = (idx0_v, idx1_v)
    tails = ((t00, t01), (t10, t11))
    ring = ((bufa, sema), (bufb, semb), (bufc, semc))

    def pair_q(p):
        return (4 * (a + 8 * p) + j) * 2

    def issue_chunk(c, q, buf, sem):
        return pltpu.async_copy(
            tbl_hbm.at[pl.ds(q, 2), pl.ds(c * _CHUNK, _LENS[c])],
            buf.at[:, pl.ds(0, _LENS[c])], sem)

    def issue_aux(q, pb):
        pltpu.async_copy(idx_hbm.at[pl.ds((q >> 6) * _B, _B)], idxs[pb],
                         semi)
        for r in range(2):
            pltpu.async_copy(tbl_hbm.at[q + r, pl.ds(_ALIGNED, _TAIL)],
                             tails[pb][r], semi)

    def wait_aux(pb):
        pltpu.make_async_copy(
            idx_hbm.at[pl.ds(0, _B)], idxs[pb], semi).wait()
        for r in range(2):
            pltpu.make_async_copy(
                tbl_hbm.at[0, pl.ds(_ALIGNED, _TAIL)], tails[pb][r],
                semi).wait()

    def wait_chunk(c, buf, sem):
        pltpu.make_async_copy(
            tbl_hbm.at[pl.ds(0, 2), pl.ds(c * _CHUNK, _LENS[c])],
            buf.at[:, pl.ds(0, _LENS[c])], sem).wait()

    def gather_pass(c, buf, pb):
        base = c * _CHUNK
        n = _LENS[c]
        idx_v = idxs[pb]

        def g_body(g, _):
            v = idx_v[pl.ds(g * 16, 16)]
            w = v - base
            m = w.astype(jnp.uint32) < jnp.uint32(n)
            if c == _NCHUNK - 1:
                wt = v - _ALIGNED
                mt = wt >= 0
            for r in range(2):
                rsplat = jnp.full((16,), r, jnp.int32)
                got = plsc.load_gather(buf, [rsplat, w], mask=m)
                if c == 0:
                    rows[r][pl.ds(g * 16, 16)] = got
                else:
                    plsc.addupdate(rows[r].at[pl.ds(g * 16, 16)], got)
                    if c == _NCHUNK - 1:
                        gt = plsc.load_gather(tails[pb][r], [wt], mask=mt)
                        plsc.addupdate(rows[r].at[pl.ds(g * 16, 16)], gt)
            return ()

        lax.fori_loop(0, _GROUPS, g_body, (), unroll=8)

    def drain_row_writes():
        for r in range(2):
            pltpu.make_async_copy(rows[r], out_hbm.at[0], semo).wait()

    def process(i, pb, more):
        q = pair_q(i)
        qn = pair_q(i + 1)
        wait_aux(pb)

        for c in range(_NCHUNK):
            buf, sem = ring[c % 3]
            wait_chunk(c, buf, sem)
            if c == 0:
                @pl.when(i > 0)
                def _drain_prev():
                    drain_row_writes()
            gather_pass(c, buf, pb)
            if c + 3 < _NCHUNK:
                issue_chunk(c + 3, q, buf, sem)
            else:
                @pl.when(more)
                def _pre(c=c, buf=buf, sem=sem):
                    issue_chunk(c + 3 - _NCHUNK, qn, buf, sem)
                    if c == 3:
                        issue_aux(qn, 1 - pb)

        pltpu.async_copy(row0_v, out_hbm.at[_CONT + q], semo)
        pltpu.async_copy(row1_v, out_hbm.at[_CONT + q + 1], semo)

    q0 = pair_q(0)
    issue_chunk(0, q0, bufa, sema)
    issue_chunk(1, q0, bufb, semb)
    issue_chunk(2, q0, bufc, semc)
    issue_aux(q0, 0)

    def pair2_body(p, _):
        process(2 * p, 0, 2 * p + 1 < _NPAIR)
        process(2 * p + 1, 1, 2 * p + 2 < _NPAIR)
        return ()

    lax.fori_loop(0, _NPAIR // 2, pair2_body, ())
    drain_row_writes()

    @pl.when(wid == _NW - 1)
    def _copy_cont():
        def cont_body(r, _):
            pltpu.sync_copy(cont_hbm.at[r], row0_v)
            pltpu.sync_copy(row0_v, out_hbm.at[r])
            return ()

        lax.fori_loop(0, _CONT, cont_body, ())


def kernel(continuous, categorical, tables):
    tbl2d = jnp.transpose(tables, (0, 2, 1)).reshape(_NF * _D, _VOCAB)
    idx_t = categorical.astype(jnp.int32).T.reshape(-1)
    cont_t = continuous.T
    out_t = _embed(tbl2d, idx_t, cont_t)
    return out_t.T


# speedup vs baseline: 1.2504x; 1.2504x over previous
import functools

import jax
import jax.numpy as jnp
from jax import lax
from jax.experimental import pallas as pl
from jax.experimental.pallas import tpu as pltpu
from jax.experimental.pallas import tpu_sc as plsc

_B = 4096
_NF = 26
_VOCAB = 100000
_D = 64
_CONT = 13
_OUT_ROWS = _CONT + _NF * _D
_NROWS = _NF * _D

_NC = 2
_NS = 16
_NW = _NC * _NS
_NPAIR = _NROWS // (2 * _NW)

_QUARTER = 25088
_LAST = 24704
_ALIGNED = 3 * _QUARTER + _LAST
_TAIL = _VOCAB - _ALIGNED
_GROUPS = _B // 16
_LENS = (_QUARTER, _QUARTER, _QUARTER, _LAST)


@functools.partial(
    pl.kernel,
    out_type=jax.ShapeDtypeStruct((_OUT_ROWS, _B), jnp.float32),
    mesh=plsc.VectorSubcoreMesh(core_axis_name="c", subcore_axis_name="s"),
    scratch_types=[
        pltpu.VMEM((2, _QUARTER), jnp.float32),
        pltpu.VMEM((2, _QUARTER), jnp.float32),
        pltpu.VMEM((_B,), jnp.int32),
        pltpu.VMEM((_B,), jnp.int32),
        pltpu.VMEM((_TAIL,), jnp.float32),
        pltpu.VMEM((_TAIL,), jnp.float32),
        pltpu.VMEM((_TAIL,), jnp.float32),
        pltpu.VMEM((_TAIL,), jnp.float32),
        pltpu.VMEM((_B,), jnp.float32),
        pltpu.VMEM((_B,), jnp.float32),
        pltpu.SemaphoreType.DMA,
        pltpu.SemaphoreType.DMA,
        pltpu.SemaphoreType.DMA,
        pltpu.SemaphoreType.DMA,
    ],
    compiler_params=pltpu.CompilerParams(
        needs_layout_passes=False, disable_bounds_checks=True,
        disable_semaphore_checks=True),
)
def _embed(tbl_hbm, idx_hbm, cont_hbm, out_hbm, bufa, bufb,
           idx0_v, idx1_v, t00, t01, t10, t11, row0_v, row1_v,
           sema, semb, semi, semo):
    wid = lax.axis_index("s") * _NC + lax.axis_index("c")
    a = wid >> 2
    j = wid & 3
    rows = (row0_v, row1_v)
    idxs = (idx0_v, idx1_v)
    tails = ((t00, t01), (t10, t11))

    def pair_q(p):
        return (4 * (a + 8 * p) + j) * 2

    def issue_chunk(c, q, buf, sem):
        return pltpu.async_copy(
            tbl_hbm.at[pl.ds(q, 2), pl.ds(c * _QUARTER, _LENS[c])],
            buf.at[:, pl.ds(0, _LENS[c])], sem)

    def issue_aux(q, pb):
        pltpu.async_copy(idx_hbm.at[pl.ds((q >> 6) * _B, _B)], idxs[pb],
                         semi)
        for r in range(2):
            pltpu.async_copy(tbl_hbm.at[q + r, pl.ds(_ALIGNED, _TAIL)],
                             tails[pb][r], semi)

    def wait_aux(pb):
        pltpu.make_async_copy(
            idx_hbm.at[pl.ds(0, _B)], idxs[pb], semi).wait()
        for r in range(2):
            pltpu.make_async_copy(
                tbl_hbm.at[0, pl.ds(_ALIGNED, _TAIL)], tails[pb][r],
                semi).wait()

    def wait_chunk(c, buf, sem):
        pltpu.make_async_copy(
            tbl_hbm.at[pl.ds(0, 2), pl.ds(c * _QUARTER, _LENS[c])],
            buf.at[:, pl.ds(0, _LENS[c])], sem).wait()

    def gather_pass(c, buf, pb):
        base = c * _QUARTER
        n = _LENS[c]
        idx_v = idxs[pb]

        def g_body(g, _):
            v = idx_v[pl.ds(g * 16, 16)]
            w = v - base
            m = w.astype(jnp.uint32) < jnp.uint32(n)
            if c == 3:
                wt = v - _ALIGNED
                mt = wt >= 0
            for r in range(2):
                rsplat = jnp.full((16,), r, jnp.int32)
                got = plsc.load_gather(buf, [rsplat, w], mask=m)
                if c == 0:
                    rows[r][pl.ds(g * 16, 16)] = got
                else:
                    plsc.addupdate(rows[r].at[pl.ds(g * 16, 16)], got)
                    if c == 3:
                        gt = plsc.load_gather(tails[pb][r], [wt], mask=mt)
                        plsc.addupdate(rows[r].at[pl.ds(g * 16, 16)], gt)
            return ()

        lax.fori_loop(0, _GROUPS, g_body, (), unroll=8)

    def drain_row_writes():
        for r in range(2):
            pltpu.make_async_copy(rows[r], out_hbm.at[0], semo).wait()

    def process(i, pb, more):
        q = pair_q(i)
        qn = pair_q(i + 1)
        wait_aux(pb)

        wait_chunk(0, bufa, sema)

        @pl.when(i > 0)
        def _drain_prev():
            drain_row_writes()

        gather_pass(0, bufa, pb)
        issue_chunk(2, q, bufa, sema)

        wait_chunk(1, bufb, semb)
        gather_pass(1, bufb, pb)
        issue_chunk(3, q, bufb, semb)

        wait_chunk(2, bufa, sema)
        gather_pass(2, bufa, pb)

        @pl.when(more)
        def _pre_a():
            issue_chunk(0, qn, bufa, sema)
            issue_aux(qn, 1 - pb)

        wait_chunk(3, bufb, semb)
        gather_pass(3, bufb, pb)

        @pl.when(more)
        def _pre_b():
            issue_chunk(1, qn, bufb, semb)

        pltpu.async_copy(row0_v, out_hbm.at[_CONT + q], semo)
        pltpu.async_copy(row1_v, out_hbm.at[_CONT + q + 1], semo)

    q0 = pair_q(0)
    issue_chunk(0, q0, bufa, sema)
    issue_chunk(1, q0, bufb, semb)
    issue_aux(q0, 0)

    def pair2_body(p, _):
        process(2 * p, 0, 2 * p + 1 < _NPAIR)
        process(2 * p + 1, 1, 2 * p + 2 < _NPAIR)
        return ()

    lax.fori_loop(0, _NPAIR // 2, pair2_body, ())
    drain_row_writes()

    @pl.when(wid == _NW - 1)
    def _copy_cont():
        def cont_body(r, _):
            pltpu.sync_copy(cont_hbm.at[r], row0_v)
            pltpu.sync_copy(row0_v, out_hbm.at[r])
            return ()

        lax.fori_loop(0, _CONT, cont_body, ())


def kernel(continuous, categorical, tables):
    tbl2d = jnp.transpose(tables, (0, 2, 1)).reshape(_NF * _D, _VOCAB)
    idx_t = categorical.astype(jnp.int32).T.reshape(-1)
    cont_t = continuous.T
    out_t = _embed(tbl2d, idx_t, cont_t)
    return out_t.T
